# baseline (speedup 1.0000x reference)
import functools

import jax
import jax.numpy as jnp
from jax import lax
from jax.experimental import pallas as pl
from jax.experimental.pallas import tpu as pltpu
from jax.experimental.pallas import tpu_sc as plsc

N = 10000
E = 320000
NC, NS = 2, 16
NW = NC * NS
CH = 128
NCHUNK = 80
E_PAD = NW * NCHUNK * CH
CH_N = 64
NCHUNK_N = 158
E_PAD_N = NW * NCHUNK_N * CH_N
AGG_ROWS = 10112
ZR = AGG_ROWS // NS
BR = 1000


def _spmm_sc(D):
    mesh = plsc.VectorSubcoreMesh(core_axis_name="c", subcore_axis_name="s")
    ch, nchunk = (CH_N, NCHUNK_N) if D >= 128 else (CH, NCHUNK)

    @functools.partial(
        pl.kernel,
        out_type=jax.ShapeDtypeStruct((NC, AGG_ROWS, D), jnp.float32),
        mesh=mesh,
        scratch_types=[
            pltpu.VMEM((nchunk, ch), jnp.int32),
            pltpu.VMEM((nchunk, ch), jnp.int32),
            pltpu.VMEM((ch, D), jnp.float32),
            pltpu.VMEM((ch, D), jnp.float32),
            pltpu.VMEM_SHARED((AGG_ROWS, D), jnp.float32),
            pltpu.SemaphoreType.DMA,
            pltpu.SemaphoreType.DMA,
            pltpu.SemaphoreType.DMA,
            pltpu.SemaphoreType.DMA,
        ],
        compiler_params=pltpu.CompilerParams(use_tc_tiling_on_sc=False),
    )
    def spmm(y_hbm, srcp_hbm, dstp_hbm, zeros_hbm, out_hbm,
             src_v, dst_v, buf0, buf1, agg, gs0, gs1, ss0, ss1):
        c = lax.axis_index("c")
        s = lax.axis_index("s")
        wid = s * NC + c
        pltpu.sync_copy(zeros_hbm.at[pl.ds(s * ZR, ZR)], agg.at[pl.ds(s * ZR, ZR)])
        pltpu.sync_copy(srcp_hbm.at[wid], src_v)
        pltpu.sync_copy(dstp_hbm.at[wid], dst_v)
        plsc.subcore_barrier()

        half = nchunk // 2

        def gath(k, buf, sem):
            return pltpu.async_copy(y_hbm.at[src_v.at[k]], buf, sem)

        def scat(k, buf, sem):
            return pltpu.async_copy(buf, agg.at[dst_v.at[k]], sem, add=True)

        def gwait(k, buf, sem):
            pltpu.make_async_copy(y_hbm.at[src_v.at[k]], buf, sem).wait()

        gath(0, buf0, gs0)
        gath(1, buf1, gs1)

        def body(j, carry):
            k = 2 * j
            gwait(k, buf0, gs0)
            d0 = scat(k, buf0, ss0)
            gwait(k + 1, buf1, gs1)
            d1 = scat(k + 1, buf1, ss1)
            d0.wait()
            gath(k + 2, buf0, gs0)
            d1.wait()
            gath(k + 3, buf1, gs1)
            return carry

        lax.fori_loop(0, half - 1, body, 0)
        gwait(nchunk - 2, buf0, gs0)
        d0 = scat(nchunk - 2, buf0, ss0)
        gwait(nchunk - 1, buf1, gs1)
        d1 = scat(nchunk - 1, buf1, ss1)
        d0.wait()
        d1.wait()
        plsc.subcore_barrier()
        pltpu.sync_copy(agg.at[pl.ds(s * ZR, ZR)], out_hbm.at[c, pl.ds(s * ZR, ZR)])

    return spmm


def _deg_sc():
    mesh = plsc.VectorSubcoreMesh(core_axis_name="c", subcore_axis_name="s")
    D = 16

    @functools.partial(
        pl.kernel,
        out_type=jax.ShapeDtypeStruct((NC, AGG_ROWS, D), jnp.float32),
        mesh=mesh,
        scratch_types=[
            pltpu.VMEM((NCHUNK, CH), jnp.int32),
            pltpu.VMEM((CH, D), jnp.float32),
            pltpu.VMEM_SHARED((AGG_ROWS, D), jnp.float32),
            pltpu.SemaphoreType.DMA,
        ],
        compiler_params=pltpu.CompilerParams(use_tc_tiling_on_sc=False),
    )
    def degk(onescol_hbm, dstp_hbm, zeros_hbm, out_hbm, dst_v, buf, agg, sem):
        c = lax.axis_index("c")
        s = lax.axis_index("s")
        wid = s * NC + c
        pltpu.sync_copy(zeros_hbm.at[pl.ds(s * ZR, ZR)], agg.at[pl.ds(s * ZR, ZR)])
        pltpu.sync_copy(onescol_hbm, buf)
        pltpu.sync_copy(dstp_hbm.at[wid], dst_v)
        plsc.subcore_barrier()

        def body(j, carry):
            ds = [pltpu.async_copy(buf, agg.at[dst_v.at[8 * j + t]], sem, add=True)
                  for t in range(8)]
            for d in ds:
                d.wait()
            return carry

        lax.fori_loop(0, NCHUNK // 8, body, 0)
        plsc.subcore_barrier()
        pltpu.sync_copy(agg.at[pl.ds(s * ZR, ZR)], out_hbm.at[c, pl.ds(s * ZR, ZR)])

    return degk


def _dot(a, b):
    return lax.dot_general(a, b, (((1,), (0,)), ((), ())),
                           precision=lax.Precision.HIGHEST,
                           preferred_element_type=jnp.float32)


def _tc_first(din, dout):
    def body(x_ref, w_ref, degs_ref, y_ref, dinv_ref):
        deg = degs_ref[0, :, 0:1] + degs_ref[1, :, 0:1] + 1.0
        dinv = lax.rsqrt(deg)
        y_ref[...] = _dot(x_ref[...], w_ref[...]) * dinv
        dinv_ref[...] = dinv

    return pl.pallas_call(
        body,
        grid=(N // BR,),
        in_specs=[
            pl.BlockSpec((BR, din), lambda i: (i, 0)),
            pl.BlockSpec((din, dout), lambda i: (0, 0)),
            pl.BlockSpec((2, BR, 16), lambda i: (0, i, 0)),
        ],
        out_specs=[
            pl.BlockSpec((BR, dout), lambda i: (i, 0)),
            pl.BlockSpec((BR, 1), lambda i: (i, 0)),
        ],
        out_shape=[
            jax.ShapeDtypeStruct((N, dout), jnp.float32),
            jax.ShapeDtypeStruct((N, 1), jnp.float32),
        ],
    )


def _tc_mid(din, dout):
    def body(raw_ref, y_ref, dinv_ref, b_ref, w_ref, out_ref):
        dinv = dinv_ref[...]
        acc = raw_ref[0] + raw_ref[1] + y_ref[...]
        X = jnp.maximum(acc * dinv + b_ref[...], 0.0)
        out_ref[...] = _dot(X, w_ref[...]) * dinv

    return pl.pallas_call(
        body,
        grid=(N // BR,),
        in_specs=[
            pl.BlockSpec((2, BR, din), lambda i: (0, i, 0)),
            pl.BlockSpec((BR, din), lambda i: (i, 0)),
            pl.BlockSpec((BR, 1), lambda i: (i, 0)),
            pl.BlockSpec((1, din), lambda i: (0, 0)),
            pl.BlockSpec((din, dout), lambda i: (0, 0)),
        ],
        out_specs=pl.BlockSpec((BR, dout), lambda i: (i, 0)),
        out_shape=jax.ShapeDtypeStruct((N, dout), jnp.float32),
    )


def _tc_last(din):
    def body(raw_ref, y_ref, dinv_ref, b_ref, out_ref):
        dinv = dinv_ref[...]
        acc = raw_ref[0] + raw_ref[1] + y_ref[...]
        X = jnp.maximum(acc * dinv + b_ref[...], 0.0)
        m = jnp.max(X, axis=1, keepdims=True)
        lse = jnp.log(jnp.sum(jnp.exp(X - m), axis=1, keepdims=True)) + m
        out_ref[...] = X - lse

    return pl.pallas_call(
        body,
        grid=(N // BR,),
        in_specs=[
            pl.BlockSpec((2, BR, din), lambda i: (0, i, 0)),
            pl.BlockSpec((BR, din), lambda i: (i, 0)),
            pl.BlockSpec((BR, 1), lambda i: (i, 0)),
            pl.BlockSpec((1, din), lambda i: (0, 0)),
        ],
        out_specs=pl.BlockSpec((BR, din), lambda i: (i, 0)),
        out_shape=jax.ShapeDtypeStruct((N, din), jnp.float32),
    )


def kernel(x, edge_index, W0, b0, W1, b1, W2, b2, W3, b3, W4, b4):
    src = edge_index[0]
    dst = edge_index[1]

    def lay(a, fill, e_pad, nchunk, ch):
        padv = jnp.full((e_pad - E,), fill, jnp.int32)
        return jnp.concatenate([a, padv]).reshape(NW, nchunk, ch)

    srcp = lay(src, 0, E_PAD, NCHUNK, CH)
    dstp = lay(dst, N, E_PAD, NCHUNK, CH)
    srcp_n = lay(src, 0, E_PAD_N, NCHUNK_N, CH_N)
    dstp_n = lay(dst, N, E_PAD_N, NCHUNK_N, CH_N)
    onescol = jnp.concatenate(
        [jnp.ones((CH, 1), jnp.float32), jnp.zeros((CH, 15), jnp.float32)], axis=1)
    zeros = {d: jnp.zeros((AGG_ROWS, d), jnp.float32) for d in (128, 64, 32, 16)}

    degp = _deg_sc()(onescol, dstp, zeros[16])
    y, dinv = _tc_first(128, 128)(x, W0, degp)

    params = [(b0, W1, 64), (b1, W2, 32), (b2, W3, 16), (b3, W4, 16)]
    din = 128
    for b, W, dout in params:
        sp, dp = (srcp_n, dstp_n) if din >= 128 else (srcp, dstp)
        raw = _spmm_sc(din)(y, sp, dp, zeros[din])
        y = _tc_mid(din, dout)(raw, y, dinv, b.reshape(1, din), W)
        din = dout
    raw = _spmm_sc(din)(y, srcp, dstp, zeros[din])
    return _tc_last(din)(raw, y, dinv, b4.reshape(1, din))


# speedup vs baseline: 1.0476x; 1.0476x over previous
import functools

import jax
import jax.numpy as jnp
from jax import lax
from jax.experimental import pallas as pl
from jax.experimental.pallas import tpu as pltpu
from jax.experimental.pallas import tpu_sc as plsc

N = 10000
E = 320000
NC, NS = 2, 16
NW = NC * NS
CH = 128
NCHUNK = 80
CH_N = 64
NCHUNK_N = 159
EW = E // NW
AGG_ROWS = 10112
ZR = AGG_ROWS // NS
BR = 1000


def _spmm_sc(D):
    mesh = plsc.VectorSubcoreMesh(core_axis_name="c", subcore_axis_name="s")
    narrow = D >= 128
    ch, nchunk = (CH_N, NCHUNK_N) if narrow else (CH, NCHUNK)
    nbuf = 3 if narrow else 2

    @functools.partial(
        pl.kernel,
        out_type=jax.ShapeDtypeStruct((NC, AGG_ROWS, D), jnp.float32),
        mesh=mesh,
        scratch_types=[
            pltpu.VMEM((nchunk, ch), jnp.int32),
            pltpu.VMEM((nchunk, ch), jnp.int32),
            [pltpu.VMEM((ch, D), jnp.float32) for _ in range(nbuf)],
            pltpu.VMEM_SHARED((AGG_ROWS, D), jnp.float32),
            [pltpu.SemaphoreType.DMA for _ in range(2 * nbuf)],
        ],
        compiler_params=pltpu.CompilerParams(use_tc_tiling_on_sc=False),
    )
    def spmm(y_hbm, srcp_hbm, dstp_hbm, zeros_hbm, out_hbm,
             src_v, dst_v, bufs, agg, sems):
        gs, ss = sems[:nbuf], sems[nbuf:]
        c = lax.axis_index("c")
        s = lax.axis_index("s")
        wid = s * NC + c
        pltpu.sync_copy(zeros_hbm.at[pl.ds(s * ZR, ZR)], agg.at[pl.ds(s * ZR, ZR)])
        pltpu.sync_copy(srcp_hbm.at[wid], src_v)
        pltpu.sync_copy(dstp_hbm.at[wid], dst_v)
        plsc.subcore_barrier()

        def gath(k, b):
            return pltpu.async_copy(y_hbm.at[src_v.at[k]], bufs[b], gs[b])

        def gwait(k, b):
            pltpu.make_async_copy(y_hbm.at[src_v.at[k]], bufs[b], gs[b]).wait()

        def scat(k, b):
            return pltpu.async_copy(bufs[b], agg.at[dst_v.at[k]], ss[b], add=True)

        if narrow:
            m = nchunk // 3
            for b in range(3):
                gath(b, b)

            def body(j, carry):
                base = 3 * j
                ds = []
                for b in range(3):
                    gwait(base + b, b)
                    ds.append(scat(base + b, b))
                for b in range(3):
                    ds[b].wait()
                    gath(base + 3 + b, b)
                return carry

            lax.fori_loop(0, m - 1, body, 0)
            base = nchunk - 3
            ds = []
            for b in range(3):
                gwait(base + b, b)
                ds.append(scat(base + b, b))
            for b in range(3):
                ds[b].wait()
        else:
            half = nchunk // 2
            gath(0, 0)

            def body(j, carry):
                k = 2 * j
                gath(k + 1, 1)
                gwait(k, 0)
                scat(k, 0).wait()

                @pl.when(j < half - 1)
                def _prefetch():
                    gath(k + 2, 0)

                gwait(k + 1, 1)
                scat(k + 1, 1).wait()
                return carry

            lax.fori_loop(0, half, body, 0)
        plsc.subcore_barrier()
        pltpu.sync_copy(agg.at[pl.ds(s * ZR, ZR)], out_hbm.at[c, pl.ds(s * ZR, ZR)])

    return spmm


def _deg_sc():
    mesh = plsc.VectorSubcoreMesh(core_axis_name="c", subcore_axis_name="s")
    D = 16

    @functools.partial(
        pl.kernel,
        out_type=jax.ShapeDtypeStruct((NC, AGG_ROWS, D), jnp.float32),
        mesh=mesh,
        scratch_types=[
            pltpu.VMEM((NCHUNK, CH), jnp.int32),
            pltpu.VMEM((CH, D), jnp.float32),
            pltpu.VMEM_SHARED((AGG_ROWS, D), jnp.float32),
            pltpu.SemaphoreType.DMA,
        ],
        compiler_params=pltpu.CompilerParams(use_tc_tiling_on_sc=False),
    )
    def degk(onescol_hbm, dstp_hbm, zeros_hbm, out_hbm, dst_v, buf, agg, sem):
        c = lax.axis_index("c")
        s = lax.axis_index("s")
        wid = s * NC + c
        pltpu.sync_copy(zeros_hbm.at[pl.ds(s * ZR, ZR)], agg.at[pl.ds(s * ZR, ZR)])
        pltpu.sync_copy(onescol_hbm, buf)
        pltpu.sync_copy(dstp_hbm.at[wid], dst_v)
        plsc.subcore_barrier()

        def body(j, carry):
            ds = [pltpu.async_copy(buf, agg.at[dst_v.at[8 * j + t]], sem, add=True)
                  for t in range(8)]
            for d in ds:
                d.wait()
            return carry

        lax.fori_loop(0, NCHUNK // 8, body, 0)
        plsc.subcore_barrier()
        pltpu.sync_copy(agg.at[pl.ds(s * ZR, ZR)], out_hbm.at[c, pl.ds(s * ZR, ZR)])

    return degk


def _dot(a, b):
    return lax.dot_general(a, b, (((1,), (0,)), ((), ())),
                           precision=lax.Precision.HIGHEST,
                           preferred_element_type=jnp.float32)


def _tc_first(din, dout):
    def body(x_ref, w_ref, degs_ref, y_ref, dinv_ref):
        deg = degs_ref[0, :, 0:1] + degs_ref[1, :, 0:1] + 1.0
        dinv = lax.rsqrt(deg)
        y_ref[...] = _dot(x_ref[...], w_ref[...]) * dinv
        dinv_ref[...] = dinv

    return pl.pallas_call(
        body,
        grid=(N // BR,),
        in_specs=[
            pl.BlockSpec((BR, din), lambda i: (i, 0)),
            pl.BlockSpec((din, dout), lambda i: (0, 0)),
            pl.BlockSpec((2, BR, 16), lambda i: (0, i, 0)),
        ],
        out_specs=[
            pl.BlockSpec((BR, dout), lambda i: (i, 0)),
            pl.BlockSpec((BR, 1), lambda i: (i, 0)),
        ],
        out_shape=[
            jax.ShapeDtypeStruct((N, dout), jnp.float32),
            jax.ShapeDtypeStruct((N, 1), jnp.float32),
        ],
    )


def _tc_mid(din, dout):
    def body(raw_ref, y_ref, dinv_ref, b_ref, w_ref, out_ref):
        dinv = dinv_ref[...]
        acc = raw_ref[0] + raw_ref[1] + y_ref[...]
        X = jnp.maximum(acc * dinv + b_ref[...], 0.0)
        out_ref[...] = _dot(X, w_ref[...]) * dinv

    return pl.pallas_call(
        body,
        grid=(N // BR,),
        in_specs=[
            pl.BlockSpec((2, BR, din), lambda i: (0, i, 0)),
            pl.BlockSpec((BR, din), lambda i: (i, 0)),
            pl.BlockSpec((BR, 1), lambda i: (i, 0)),
            pl.BlockSpec((1, din), lambda i: (0, 0)),
            pl.BlockSpec((din, dout), lambda i: (0, 0)),
        ],
        out_specs=pl.BlockSpec((BR, dout), lambda i: (i, 0)),
        out_shape=jax.ShapeDtypeStruct((N, dout), jnp.float32),
    )


def _tc_last(din):
    def body(raw_ref, y_ref, dinv_ref, b_ref, out_ref):
        dinv = dinv_ref[...]
        acc = raw_ref[0] + raw_ref[1] + y_ref[...]
        X = jnp.maximum(acc * dinv + b_ref[...], 0.0)
        m = jnp.max(X, axis=1, keepdims=True)
        lse = jnp.log(jnp.sum(jnp.exp(X - m), axis=1, keepdims=True)) + m
        out_ref[...] = X - lse

    return pl.pallas_call(
        body,
        grid=(N // BR,),
        in_specs=[
            pl.BlockSpec((2, BR, din), lambda i: (0, i, 0)),
            pl.BlockSpec((BR, din), lambda i: (i, 0)),
            pl.BlockSpec((BR, 1), lambda i: (i, 0)),
            pl.BlockSpec((1, din), lambda i: (0, 0)),
        ],
        out_specs=pl.BlockSpec((BR, din), lambda i: (i, 0)),
        out_shape=jax.ShapeDtypeStruct((N, din), jnp.float32),
    )


def kernel(x, edge_index, W0, b0, W1, b1, W2, b2, W3, b3, W4, b4):
    src = edge_index[0]
    dst = edge_index[1]

    def lay(a, is_dst, nchunk, ch):
        npad = nchunk * ch - EW
        if is_dst:
            padv = N + (jnp.arange(npad, dtype=jnp.int32) % (AGG_ROWS - N))
        else:
            padv = jnp.zeros((npad,), jnp.int32)
        padv = jnp.broadcast_to(padv, (NW, npad))
        return jnp.concatenate([a.reshape(NW, EW), padv], axis=1).reshape(NW, nchunk, ch)

    srcp = lay(src, False, NCHUNK, CH)
    dstp = lay(dst, True, NCHUNK, CH)
    srcp_n = lay(src, False, NCHUNK_N, CH_N)
    dstp_n = lay(dst, True, NCHUNK_N, CH_N)
    onescol = jnp.concatenate(
        [jnp.ones((CH, 1), jnp.float32), jnp.zeros((CH, 15), jnp.float32)], axis=1)
    zeros = {d: jnp.zeros((AGG_ROWS, d), jnp.float32) for d in (128, 64, 32, 16)}

    degp = _deg_sc()(onescol, dstp, zeros[16])
    y, dinv = _tc_first(128, 128)(x, W0, degp)

    params = [(b0, W1, 64), (b1, W2, 32), (b2, W3, 16), (b3, W4, 16)]
    din = 128
    for b, W, dout in params:
        sp, dp = (srcp_n, dstp_n) if din >= 128 else (srcp, dstp)
        raw = _spmm_sc(din)(y, sp, dp, zeros[din])
        y = _tc_mid(din, dout)(raw, y, dinv, b.reshape(1, din), W)
        din = dout
    raw = _spmm_sc(din)(y, srcp, dstp, zeros[din])
    return _tc_last(din)(raw, y, dinv, b4.reshape(1, din))


# speedup vs baseline: 1.1701x; 1.1170x over previous
import functools

import jax
import jax.numpy as jnp
from jax import lax
from jax.experimental import pallas as pl
from jax.experimental.pallas import tpu as pltpu
from jax.experimental.pallas import tpu_sc as plsc

N = 10000
E = 320000
NC, NS = 2, 16
NW = NC * NS
CH = 128
NCHUNK = 80
SLOTS_W = 10240
CH_N = 64
NCHUNK_N = 158
SLOTS_N = 10112
EW = E // NW
AGG_ROWS = 10112
ZR = AGG_ROWS // NS
BR = 1000


def _spmm_sc(D):
    mesh = plsc.VectorSubcoreMesh(core_axis_name="c", subcore_axis_name="s")
    narrow = D >= 128
    nbuf = 2
    if narrow:
        idx_shape = (NCHUNK_N, CH_N)
        buf_rows = CH_N
        nchunk = NCHUNK_N
    else:
        BK = 32768 // D
        nchunk = SLOTS_W // BK
        idx_shape = (nchunk, BK)
        buf_rows = BK

    @functools.partial(
        pl.kernel,
        out_type=jax.ShapeDtypeStruct((NC, AGG_ROWS, D), jnp.float32),
        mesh=mesh,
        scratch_types=[
            pltpu.VMEM(idx_shape, jnp.int32),
            pltpu.VMEM(idx_shape, jnp.int32),
            [pltpu.VMEM((buf_rows, D), jnp.float32) for _ in range(nbuf)],
            pltpu.VMEM_SHARED((AGG_ROWS, D), jnp.float32),
            [pltpu.SemaphoreType.DMA for _ in range(2 * nbuf)],
        ],
        compiler_params=pltpu.CompilerParams(use_tc_tiling_on_sc=False),
    )
    def spmm(y_hbm, srcp_hbm, dstp_hbm, zeros_hbm, out_hbm,
             src_v, dst_v, bufs, agg, sems):
        gs, ss = sems[:nbuf], sems[nbuf:]
        c = lax.axis_index("c")
        s = lax.axis_index("s")
        wid = s * NC + c
        pltpu.sync_copy(zeros_hbm.at[pl.ds(s * ZR, ZR)], agg.at[pl.ds(s * ZR, ZR)])
        pltpu.sync_copy(srcp_hbm.at[wid], src_v)
        pltpu.sync_copy(dstp_hbm.at[wid], dst_v)
        plsc.subcore_barrier()

        def gath(k, b):
            return pltpu.async_copy(y_hbm.at[src_v.at[k]], bufs[b], gs[b])

        def gwait(k, b):
            pltpu.make_async_copy(y_hbm.at[src_v.at[k]], bufs[b], gs[b]).wait()

        def scat(k, b):
            return pltpu.async_copy(bufs[b], agg.at[dst_v.at[k]], ss[b], add=True)

        if narrow:
            half = nchunk // 2
            gath(0, 0)
            gath(1, 1)

            def body(j, carry):
                k = 2 * j
                gwait(k, 0)
                d0 = scat(k, 0)
                gwait(k + 1, 1)
                d1 = scat(k + 1, 1)
                d0.wait()
                gath(k + 2, 0)
                d1.wait()
                gath(k + 3, 1)
                return carry

            lax.fori_loop(0, half - 1, body, 0)
            gwait(nchunk - 2, 0)
            d0 = scat(nchunk - 2, 0)
            gwait(nchunk - 1, 1)
            d1 = scat(nchunk - 1, 1)
            d0.wait()
            d1.wait()
        else:
            gath(0, 0)
            for k in range(nchunk):
                b = k & 1
                if k + 1 < nchunk:
                    gath(k + 1, 1 - b)
                gwait(k, b)
                scat(k, b).wait()
        plsc.subcore_barrier()
        pltpu.sync_copy(agg.at[pl.ds(s * ZR, ZR)], out_hbm.at[c, pl.ds(s * ZR, ZR)])

    return spmm


def _deg_sc():
    mesh = plsc.VectorSubcoreMesh(core_axis_name="c", subcore_axis_name="s")
    D = 16

    @functools.partial(
        pl.kernel,
        out_type=jax.ShapeDtypeStruct((NC, AGG_ROWS, D), jnp.float32),
        mesh=mesh,
        scratch_types=[
            pltpu.VMEM((NCHUNK, CH), jnp.int32),
            pltpu.VMEM((CH, D), jnp.float32),
            pltpu.VMEM_SHARED((AGG_ROWS, D), jnp.float32),
            pltpu.SemaphoreType.DMA,
        ],
        compiler_params=pltpu.CompilerParams(use_tc_tiling_on_sc=False),
    )
    def degk(onescol_hbm, dstp_hbm, zeros_hbm, out_hbm, dst_v, buf, agg, sem):
        c = lax.axis_index("c")
        s = lax.axis_index("s")
        wid = s * NC + c
        pltpu.sync_copy(zeros_hbm.at[pl.ds(s * ZR, ZR)], agg.at[pl.ds(s * ZR, ZR)])
        pltpu.sync_copy(onescol_hbm, buf)
        pltpu.sync_copy(dstp_hbm.at[wid], dst_v)
        plsc.subcore_barrier()

        def body(j, carry):
            ds = [pltpu.async_copy(buf, agg.at[dst_v.at[8 * j + t]], sem, add=True)
                  for t in range(8)]
            for d in ds:
                d.wait()
            return carry

        nfull = NCHUNK // 8
        lax.fori_loop(0, nfull, body, 0)
        tail = [pltpu.async_copy(buf, agg.at[dst_v.at[8 * nfull + t]], sem, add=True)
                for t in range(NCHUNK - 8 * nfull)]
        for d in tail:
            d.wait()
        plsc.subcore_barrier()
        pltpu.sync_copy(agg.at[pl.ds(s * ZR, ZR)], out_hbm.at[c, pl.ds(s * ZR, ZR)])

    return degk


def _dot(a, b):
    return lax.dot_general(a, b, (((1,), (0,)), ((), ())),
                           precision=lax.Precision.HIGHEST,
                           preferred_element_type=jnp.float32)


def _tc_first(din, dout):
    def body(x_ref, w_ref, degs_ref, y_ref, dinv_ref):
        deg = degs_ref[0, :, 0:1] + degs_ref[1, :, 0:1] + 1.0
        dinv = lax.rsqrt(deg)
        y_ref[...] = _dot(x_ref[...], w_ref[...]) * dinv
        dinv_ref[...] = dinv

    return pl.pallas_call(
        body,
        grid=(N // BR,),
        in_specs=[
            pl.BlockSpec((BR, din), lambda i: (i, 0)),
            pl.BlockSpec((din, dout), lambda i: (0, 0)),
            pl.BlockSpec((2, BR, 16), lambda i: (0, i, 0)),
        ],
        out_specs=[
            pl.BlockSpec((BR, dout), lambda i: (i, 0)),
            pl.BlockSpec((BR, 1), lambda i: (i, 0)),
        ],
        out_shape=[
            jax.ShapeDtypeStruct((N, dout), jnp.float32),
            jax.ShapeDtypeStruct((N, 1), jnp.float32),
        ],
    )


def _tc_mid(din, dout):
    def body(raw_ref, y_ref, dinv_ref, b_ref, w_ref, out_ref):
        dinv = dinv_ref[...]
        acc = raw_ref[0] + raw_ref[1] + y_ref[...]
        X = jnp.maximum(acc * dinv + b_ref[...], 0.0)
        out_ref[...] = _dot(X, w_ref[...]) * dinv

    return pl.pallas_call(
        body,
        grid=(N // BR,),
        in_specs=[
            pl.BlockSpec((2, BR, din), lambda i: (0, i, 0)),
            pl.BlockSpec((BR, din), lambda i: (i, 0)),
            pl.BlockSpec((BR, 1), lambda i: (i, 0)),
            pl.BlockSpec((1, din), lambda i: (0, 0)),
            pl.BlockSpec((din, dout), lambda i: (0, 0)),
        ],
        out_specs=pl.BlockSpec((BR, dout), lambda i: (i, 0)),
        out_shape=jax.ShapeDtypeStruct((N, dout), jnp.float32),
    )


def _tc_last(din):
    def body(raw_ref, y_ref, dinv_ref, b_ref, out_ref):
        dinv = dinv_ref[...]
        acc = raw_ref[0] + raw_ref[1] + y_ref[...]
        X = jnp.maximum(acc * dinv + b_ref[...], 0.0)
        m = jnp.max(X, axis=1, keepdims=True)
        lse = jnp.log(jnp.sum(jnp.exp(X - m), axis=1, keepdims=True)) + m
        out_ref[...] = X - lse

    return pl.pallas_call(
        body,
        grid=(N // BR,),
        in_specs=[
            pl.BlockSpec((2, BR, din), lambda i: (0, i, 0)),
            pl.BlockSpec((BR, din), lambda i: (i, 0)),
            pl.BlockSpec((BR, 1), lambda i: (i, 0)),
            pl.BlockSpec((1, din), lambda i: (0, 0)),
        ],
        out_specs=pl.BlockSpec((BR, din), lambda i: (i, 0)),
        out_shape=jax.ShapeDtypeStruct((N, din), jnp.float32),
    )


def kernel(x, edge_index, W0, b0, W1, b1, W2, b2, W3, b3, W4, b4):
    src = edge_index[0]
    dst = edge_index[1]

    def lay(a, is_dst, slots):
        npad = slots - EW
        if is_dst:
            s_idx = (jnp.arange(NW, dtype=jnp.int32) // NC)[:, None]
            padv = N + 7 * s_idx + (jnp.arange(npad, dtype=jnp.int32) % 7)[None, :]
        else:
            padv = jnp.zeros((NW, npad), jnp.int32)
        return jnp.concatenate([a.reshape(NW, EW), padv], axis=1)

    src_w = lay(src, False, SLOTS_W)
    dst_w = lay(dst, True, SLOTS_W)
    src_n = lay(src, False, SLOTS_N)
    dst_n = lay(dst, True, SLOTS_N)
    dstp = dst_w.reshape(NW, NCHUNK, CH)
    srcp_n = src_n.reshape(NW, NCHUNK_N, CH_N)
    dstp_n = dst_n.reshape(NW, NCHUNK_N, CH_N)

    def wide(a, D):
        bk = 32768 // D
        return a.reshape(NW, SLOTS_W // bk, bk)
    onescol = jnp.concatenate(
        [jnp.ones((CH, 1), jnp.float32), jnp.zeros((CH, 15), jnp.float32)], axis=1)
    zeros = {d: jnp.zeros((AGG_ROWS, d), jnp.float32) for d in (128, 64, 32, 16)}

    degp = _deg_sc()(onescol, dstp, zeros[16])
    y, dinv = _tc_first(128, 128)(x, W0, degp)

    params = [(b0, W1, 64), (b1, W2, 32), (b2, W3, 16), (b3, W4, 16)]
    din = 128
    for b, W, dout in params:
        if din >= 128:
            sp, dp = srcp_n, dstp_n
        else:
            sp, dp = wide(src_w, din), wide(dst_w, din)
        raw = _spmm_sc(din)(y, sp, dp, zeros[din])
        y = _tc_mid(din, dout)(raw, y, dinv, b.reshape(1, din), W)
        din = dout
    raw = _spmm_sc(din)(y, wide(src_w, din), wide(dst_w, din), zeros[din])
    return _tc_last(din)(raw, y, dinv, b4.reshape(1, din))
